# Initial kernel scaffold; baseline (speedup 1.0000x reference)
import jax
import jax.numpy as jnp
from jax.experimental import pallas as pl


def kernel(x, edge_index, edge_weight, Wq, bq, Wk, bk, Wv, bv, Wskip, bskip, Wsl, bsl, Wsr, Wrel, brel, Wroot, Wqkv, bqkv, Wproj, bproj, Wfc, bfc):
    raise NotImplementedError("write your pallas kernel here")


# speedup vs baseline: 1.4336x; 1.4336x over previous
import dataclasses
import functools

import jax
import jax.numpy as jnp
from jax import lax
from jax.experimental import pallas as pl
from jax.experimental.pallas import tpu as pltpu
from jax.experimental.pallas import tpu_sc as plsc

N = 10000
E = 320000
D = 128
H = 2
HD = D // H

NC = 2
NS = 16
L = 16
NW = NC * NS

C = 128
NCHUNK = 79
EPW = C * NCHUNK
EPAD = EPW * NW
NPAD = 10240
RPT = NPAD // NS

_mesh = plsc.VectorSubcoreMesh(core_axis_name="c", subcore_axis_name="s")

_sc_params = pltpu.CompilerParams()
if "needs_layout_passes" in pltpu.CompilerParams.__dataclass_fields__:
    _sc_params = dataclasses.replace(_sc_params, needs_layout_passes=False)


def _wid():
    return lax.axis_index("s") * NC + lax.axis_index("c")


@functools.partial(
    pl.kernel,
    out_type=(
        jax.ShapeDtypeStruct((NW, NCHUNK, C), jnp.float32),
        jax.ShapeDtypeStruct((NW, L), jnp.float32),
    ),
    mesh=_mesh,
    compiler_params=_sc_params,
    scratch_types=[
        pltpu.VMEM((C,), jnp.int32),
        pltpu.VMEM((C,), jnp.int32),
        pltpu.VMEM((C, D), jnp.float32),
        pltpu.VMEM((C, D), jnp.float32),
        pltpu.VMEM((C,), jnp.float32),
        pltpu.VMEM((L,), jnp.float32),
    ],
)
def _sc_alpha(q_hbm, k_hbm, src_hbm, dst_hbm, alpha_hbm, mx_hbm,
              si, di, qr, kr, al, mx):
    wid = _wid()
    inv_sqrt_d = 1.0 / (D ** 0.5)
    lanes = lax.iota(jnp.int32, L)
    mx[...] = jnp.full((L,), -1e30, jnp.float32)

    @pl.loop(0, NCHUNK)
    def _chunk(j):
        pltpu.sync_copy(src_hbm.at[wid, j], si)
        pltpu.sync_copy(dst_hbm.at[wid, j], di)
        pltpu.sync_copy(q_hbm.at[di], qr)
        pltpu.sync_copy(k_hbm.at[si], kr)

        rows = [lanes + g * L for g in range(C // L)]

        def dot_body(d, accs):
            cold = jnp.full((L,), d, jnp.int32)
            return tuple(
                acc + plsc.load_gather(qr, [rows[g], cold])
                * plsc.load_gather(kr, [rows[g], cold])
                for g, acc in enumerate(accs)
            )

        accs = lax.fori_loop(
            0, D, dot_body,
            tuple(jnp.zeros((L,), jnp.float32) for _ in range(C // L)))
        for g in range(C // L):
            a_g = accs[g] * inv_sqrt_d
            al[pl.ds(g * L, L)] = a_g
            mx[...] = jnp.maximum(mx[...], a_g)
        pltpu.sync_copy(al, alpha_hbm.at[wid, j])

    pltpu.sync_copy(mx, mx_hbm.at[wid])


@functools.partial(
    pl.kernel,
    out_type=jax.ShapeDtypeStruct((NC, NPAD, D), jnp.float32),
    mesh=_mesh,
    compiler_params=_sc_params,
    scratch_types=[
        pltpu.VMEM((C,), jnp.int32),
        pltpu.VMEM((C,), jnp.int32),
        pltpu.VMEM((C, D), jnp.float32),
        pltpu.VMEM((C, D), jnp.float32),
        pltpu.VMEM((C,), jnp.float32),
        pltpu.VMEM((NW, L), jnp.float32),
        pltpu.VMEM_SHARED((NPAD, D), jnp.float32),
    ],
)
def _sc_msg_v(v_hbm, src_hbm, dst_hbm, alpha_hbm, mx_hbm, accv_hbm,
              si, di, vr, sv, al, mxv, accv_s):
    cid = lax.axis_index("c")
    sid = lax.axis_index("s")
    wid = sid * NC + cid
    lanes = lax.iota(jnp.int32, L)
    rows = [lanes + g * L for g in range(C // L)]

    @pl.loop(0, C)
    def _zrow(r):
        for g in range(D // L):
            sv[r, pl.ds(g * L, L)] = jnp.zeros((L,), jnp.float32)

    base = sid * RPT
    for t in range(RPT // C):
        pltpu.sync_copy(sv, accv_s.at[pl.ds(base + t * C, C)])

    pltpu.sync_copy(mx_hbm, mxv)

    def mx_body(i, mv):
        return jnp.maximum(mv, mxv[i, :])

    m = jnp.max(lax.fori_loop(1, NW, mx_body, mxv[0, :]))

    plsc.subcore_barrier()

    @pl.loop(0, NCHUNK)
    def _chunk(j):
        pltpu.sync_copy(src_hbm.at[wid, j], si)
        pltpu.sync_copy(dst_hbm.at[wid, j], di)
        pltpu.sync_copy(alpha_hbm.at[wid, j], al)
        pltpu.sync_copy(v_hbm.at[si], vr)

        exs = [jnp.exp(al[pl.ds(g * L, L)] - m) for g in range(C // L)]

        def scale_body(d, carry):
            cold = jnp.full((L,), d, jnp.int32)
            for g in range(C // L):
                col = plsc.load_gather(vr, [rows[g], cold])
                plsc.store_scatter(sv, [rows[g], cold], col * exs[g])
            return carry

        lax.fori_loop(0, D, scale_body, 0)
        pltpu.sync_copy(sv, accv_s.at[di], add=True)

    plsc.subcore_barrier()
    pltpu.sync_copy(accv_s.at[pl.ds(base, RPT)], accv_hbm.at[cid, pl.ds(base, RPT)])


@functools.partial(
    pl.kernel,
    out_type=jax.ShapeDtypeStruct((NC, NPAD, D), jnp.float32),
    mesh=_mesh,
    compiler_params=_sc_params,
    scratch_types=[
        pltpu.VMEM((C,), jnp.int32),
        pltpu.VMEM((C,), jnp.float32),
        pltpu.VMEM((C, D), jnp.float32),
        pltpu.VMEM((NW, L), jnp.float32),
        pltpu.VMEM_SHARED((NPAD, D), jnp.float32),
    ],
)
def _sc_aux(dst_hbm, alpha_hbm, mx_hbm, aux_hbm, di, al, auxb, mxv, aux_s):
    cid = lax.axis_index("c")
    sid = lax.axis_index("s")
    wid = sid * NC + cid
    lanes = lax.iota(jnp.int32, L)
    rows = [lanes + g * L for g in range(C // L)]

    @pl.loop(0, C)
    def _zrow(r):
        for g in range(D // L):
            auxb[r, pl.ds(g * L, L)] = jnp.zeros((L,), jnp.float32)

    base = sid * RPT
    for t in range(RPT // C):
        pltpu.sync_copy(auxb, aux_s.at[pl.ds(base + t * C, C)])
    ones = jnp.ones((L,), jnp.float32)
    col1 = jnp.full((L,), 1, jnp.int32)
    col0 = jnp.full((L,), 0, jnp.int32)
    for g in range(C // L):
        plsc.store_scatter(auxb, [rows[g], col1], ones)

    pltpu.sync_copy(mx_hbm, mxv)

    def mx_body(i, mv):
        return jnp.maximum(mv, mxv[i, :])

    m = jnp.max(lax.fori_loop(1, NW, mx_body, mxv[0, :]))

    plsc.subcore_barrier()

    @pl.loop(0, NCHUNK)
    def _chunk(j):
        pltpu.sync_copy(dst_hbm.at[wid, j], di)
        pltpu.sync_copy(alpha_hbm.at[wid, j], al)
        for g in range(C // L):
            ex_g = jnp.exp(al[pl.ds(g * L, L)] - m)
            plsc.store_scatter(auxb, [rows[g], col0], ex_g)
        pltpu.sync_copy(auxb, aux_s.at[di], add=True)

    plsc.subcore_barrier()
    pltpu.sync_copy(aux_s.at[pl.ds(base, RPT)], aux_hbm.at[cid, pl.ds(base, RPT)])


@functools.partial(
    pl.kernel,
    out_type=jax.ShapeDtypeStruct((NC, NPAD, D), jnp.float32),
    mesh=_mesh,
    compiler_params=_sc_params,
    scratch_types=[
        pltpu.VMEM((C,), jnp.int32),
        pltpu.VMEM((C,), jnp.int32),
        pltpu.VMEM((C, D), jnp.float32),
        pltpu.VMEM_SHARED((NPAD, D), jnp.float32),
    ],
)
def _sc_msg_x1(x_hbm, src_hbm, dst_hbm, acc_hbm, si, di, xr, acc_s):
    cid = lax.axis_index("c")
    sid = lax.axis_index("s")
    wid = sid * NC + cid

    @pl.loop(0, C)
    def _zrow(r):
        for g in range(D // L):
            xr[r, pl.ds(g * L, L)] = jnp.zeros((L,), jnp.float32)

    base = sid * RPT
    for t in range(RPT // C):
        pltpu.sync_copy(xr, acc_s.at[pl.ds(base + t * C, C)])

    plsc.subcore_barrier()

    @pl.loop(0, NCHUNK)
    def _chunk(j):
        pltpu.sync_copy(src_hbm.at[wid, j], si)
        pltpu.sync_copy(dst_hbm.at[wid, j], di)
        pltpu.sync_copy(x_hbm.at[si], xr)
        pltpu.sync_copy(xr, acc_s.at[di], add=True)

    plsc.subcore_barrier()
    pltpu.sync_copy(acc_s.at[pl.ds(base, RPT)], acc_hbm.at[cid, pl.ds(base, RPT)])


@functools.partial(
    pl.kernel,
    out_type=jax.ShapeDtypeStruct((NC, NPAD, D), jnp.float32),
    mesh=_mesh,
    compiler_params=_sc_params,
    scratch_types=[
        pltpu.VMEM((C,), jnp.int32),
        pltpu.VMEM((C,), jnp.int32),
        pltpu.VMEM((C,), jnp.float32),
        pltpu.VMEM((C, D), jnp.float32),
        pltpu.VMEM((C, D), jnp.float32),
        pltpu.VMEM_SHARED((NPAD, D), jnp.float32),
    ],
)
def _sc_msg_x2(x_hbm, src_hbm, dst_hbm, ew_hbm, acc_hbm,
               si, di, ew, xr, sx, acc_s):
    cid = lax.axis_index("c")
    sid = lax.axis_index("s")
    wid = sid * NC + cid
    lanes = lax.iota(jnp.int32, L)
    rows = [lanes + g * L for g in range(C // L)]

    @pl.loop(0, C)
    def _zrow(r):
        for g in range(D // L):
            sx[r, pl.ds(g * L, L)] = jnp.zeros((L,), jnp.float32)

    base = sid * RPT
    for t in range(RPT // C):
        pltpu.sync_copy(sx, acc_s.at[pl.ds(base + t * C, C)])

    plsc.subcore_barrier()

    @pl.loop(0, NCHUNK)
    def _chunk(j):
        pltpu.sync_copy(src_hbm.at[wid, j], si)
        pltpu.sync_copy(dst_hbm.at[wid, j], di)
        pltpu.sync_copy(ew_hbm.at[wid, j], ew)
        pltpu.sync_copy(x_hbm.at[si], xr)

        ews = [ew[pl.ds(g * L, L)] for g in range(C // L)]

        def scale_body(d, carry):
            cold = jnp.full((L,), d, jnp.int32)
            for g in range(C // L):
                col = plsc.load_gather(xr, [rows[g], cold])
                plsc.store_scatter(sx, [rows[g], cold], col * ews[g])
            return carry

        lax.fori_loop(0, D, scale_body, 0)
        pltpu.sync_copy(sx, acc_s.at[di], add=True)

    plsc.subcore_barrier()
    pltpu.sync_copy(acc_s.at[pl.ds(base, RPT)], acc_hbm.at[cid, pl.ds(base, RPT)])


def _tc_qkv_body(x_ref, w_ref, b_ref, q_ref, k_ref, v_ref):
    y = jnp.dot(x_ref[...], w_ref[...], preferred_element_type=jnp.float32)
    y = y + b_ref[...]
    q_ref[...] = y[:, :D]
    k_ref[...] = y[:, D:2 * D]
    v_ref[...] = y[:, 2 * D:]


def _tc_qkv(xp, W3, b3):
    blk = 1024
    grid = (NPAD // blk,)
    return pl.pallas_call(
        _tc_qkv_body,
        grid=grid,
        in_specs=[
            pl.BlockSpec((blk, D), lambda i: (i, 0)),
            pl.BlockSpec((D, 3 * D), lambda i: (0, 0)),
            pl.BlockSpec((1, 3 * D), lambda i: (0, 0)),
        ],
        out_specs=[
            pl.BlockSpec((blk, D), lambda i: (i, 0)),
            pl.BlockSpec((blk, D), lambda i: (i, 0)),
            pl.BlockSpec((blk, D), lambda i: (i, 0)),
        ],
        out_shape=[jax.ShapeDtypeStruct((NPAD, D), jnp.float32)] * 3,
    )(xp, W3, b3)


def _tc_final_body(x_ref, accv_ref, aux_ref, accx_ref, accxw_ref,
                   wskip_ref, bskip_ref, wsl_ref, bsl_ref, wsr_ref,
                   wrel_ref, brel_ref, wroot_ref,
                   wqkv_ref, bqkv_ref, wproj_ref, bproj_ref,
                   wfc_ref, bfc_ref, out_ref):
    x = x_ref[...]
    accv = accv_ref[0] + accv_ref[1]
    aux = aux_ref[0] + aux_ref[1]
    denom = aux[:, 0:1]
    deg_c = jnp.maximum(aux[:, 1:2], 1.0)
    accx = accx_ref[0] + accx_ref[1]
    accxw = accxw_ref[0] + accxw_ref[1]

    def mm(a, w):
        return jnp.dot(a, w, preferred_element_type=jnp.float32)

    x_gc1 = accv / (denom + 1e-16) + mm(x, wskip_ref[...]) + bskip_ref[...]
    x_gc2 = mm(accx / deg_c, wsl_ref[...]) + bsl_ref[...] + mm(x, wsr_ref[...])
    x_gc3 = mm(accxw / deg_c, wrel_ref[...]) + brel_ref[...] + mm(x, wroot_ref[...])

    toks = [x_gc1, x_gc2, x_gc3]
    wqkv = wqkv_ref[...]
    bqkv = bqkv_ref[...]
    qs, ks, vs = [], [], []
    for t in toks:
        qkv = mm(t, wqkv) + bqkv
        qs.append(qkv[:, :D])
        ks.append(qkv[:, D:2 * D])
        vs.append(qkv[:, 2 * D:])

    scale = HD ** -0.5
    outs = []
    for i in range(3):
        halves = []
        for h in range(H):
            sl = slice(h * HD, (h + 1) * HD)
            s = [jnp.sum(qs[i][:, sl] * ks[j][:, sl], axis=1, keepdims=True)
                 * scale for j in range(3)]
            m = jnp.maximum(jnp.maximum(s[0], s[1]), s[2])
            e = [jnp.exp(sj - m) for sj in s]
            den = e[0] + e[1] + e[2]
            halves.append(sum(e[j] / den * vs[j][:, sl] for j in range(3)))
        out_i = jnp.concatenate(halves, axis=1)
        outs.append(mm(out_i, wproj_ref[...]) + bproj_ref[...])

    wfc = wfc_ref[...]
    final = mm(outs[0], wfc[:D, :]) + mm(outs[1], wfc[D:2 * D, :]) \
        + mm(outs[2], wfc[2 * D:, :]) + bfc_ref[...]
    out_ref[...] = final


def _tc_final(xp, accv, aux, accx, accxw, Wskip, bskip, Wsl, bsl, Wsr,
              Wrel, brel, Wroot, Wqkv, bqkv, Wproj, bproj, Wfc, bfc):
    bskip, bsl, brel, bproj, bfc = (
        b.reshape(1, D) for b in (bskip, bsl, brel, bproj, bfc))
    bqkv = bqkv.reshape(1, 3 * D)
    blk = 1024
    grid = (NPAD // blk,)

    def row_spec(minor):
        return pl.BlockSpec((NC, blk, minor), lambda i: (0, i, 0))

    def w_spec(r, c):
        return pl.BlockSpec((r, c), lambda i: (0, 0))

    return pl.pallas_call(
        _tc_final_body,
        grid=grid,
        in_specs=[
            pl.BlockSpec((blk, D), lambda i: (i, 0)),
            row_spec(D), row_spec(D), row_spec(D), row_spec(D),
            w_spec(D, D), w_spec(1, D),
            w_spec(D, D), w_spec(1, D), w_spec(D, D),
            w_spec(D, D), w_spec(1, D), w_spec(D, D),
            w_spec(D, 3 * D), w_spec(1, 3 * D),
            w_spec(D, D), w_spec(1, D),
            w_spec(3 * D, D), w_spec(1, D),
        ],
        out_specs=pl.BlockSpec((blk, D), lambda i: (i, 0)),
        out_shape=jax.ShapeDtypeStruct((NPAD, D), jnp.float32),
    )(xp, accv, aux, accx, accxw, Wskip, bskip, Wsl, bsl, Wsr,
      Wrel, brel, Wroot, Wqkv, bqkv, Wproj, bproj, Wfc, bfc)


def kernel(x, edge_index, edge_weight, Wq, bq, Wk, bk, Wv, bv, Wskip, bskip,
           Wsl, bsl, Wsr, Wrel, brel, Wroot, Wqkv, bqkv, Wproj, bproj,
           Wfc, bfc):
    xp = jnp.pad(x, ((0, NPAD - N), (0, 0)))
    pad_e = EPAD - E
    src = jnp.concatenate([edge_index[0], jnp.full((pad_e,), N, jnp.int32)])
    dst = jnp.concatenate([edge_index[1], jnp.full((pad_e,), N, jnp.int32)])
    ew = jnp.concatenate([edge_weight, jnp.zeros((pad_e,), jnp.float32)])
    srcr = src.reshape(NW, NCHUNK, C)
    dstr = dst.reshape(NW, NCHUNK, C)
    ewr = ew.reshape(NW, NCHUNK, C)

    W3 = jnp.concatenate([Wq, Wk, Wv], axis=1)
    b3 = jnp.concatenate([bq, bk, bv]).reshape(1, 3 * D)

    q, k, v = _tc_qkv(xp, W3, b3)
    alpha, mx = _sc_alpha(q, k, srcr, dstr)
    accv = _sc_msg_v(v, srcr, dstr, alpha, mx)
    aux = _sc_aux(dstr, alpha, mx)
    accx = _sc_msg_x1(xp, srcr, dstr)
    accxw = _sc_msg_x2(xp, srcr, dstr, ewr)

    out = _tc_final(xp, accv, aux, accx, accxw, Wskip, bskip, Wsl, bsl, Wsr,
                    Wrel, brel, Wroot, Wqkv, bqkv, Wproj, bproj, Wfc, bfc)
    return out[:N]


# speedup vs baseline: 3.7381x; 2.6075x over previous
import dataclasses
import functools

import jax
import jax.numpy as jnp
from jax import lax
from jax.experimental import pallas as pl
from jax.experimental.pallas import tpu as pltpu
from jax.experimental.pallas import tpu_sc as plsc

N = 10000
E = 320000
D = 128
H = 2
HD = D // H

NC = 2
NS = 16
L = 16
NW = NC * NS

C = 128
NCHUNK = 79
EPW = C * NCHUNK
EPAD = EPW * NW
NPAD = 10240
RPT = NPAD // NS

_mesh = plsc.VectorSubcoreMesh(core_axis_name="c", subcore_axis_name="s")

_sc_params = pltpu.CompilerParams()
if "needs_layout_passes" in pltpu.CompilerParams.__dataclass_fields__:
    _sc_params = dataclasses.replace(_sc_params, needs_layout_passes=False)


def _wid():
    return lax.axis_index("s") * NC + lax.axis_index("c")


@functools.partial(
    pl.kernel,
    out_type=(
        jax.ShapeDtypeStruct((NW, NCHUNK, C), jnp.float32),
        jax.ShapeDtypeStruct((NW, L), jnp.float32),
    ),
    mesh=_mesh,
    compiler_params=_sc_params,
    scratch_types=[
        pltpu.VMEM((C,), jnp.int32),
        pltpu.VMEM((C,), jnp.int32),
        pltpu.VMEM((C, D), jnp.float32),
        pltpu.VMEM((C, D), jnp.float32),
        pltpu.VMEM((C,), jnp.float32),
        pltpu.VMEM((L,), jnp.float32),
    ],
)
def _sc_alpha(q_hbm, k_hbm, src_hbm, dst_hbm, alpha_hbm, mx_hbm,
              si, di, qr, kr, al, mx):
    wid = _wid()
    inv_sqrt_d = 1.0 / (D ** 0.5)
    lanes = lax.iota(jnp.int32, L)
    mx[...] = jnp.full((L,), -1e30, jnp.float32)

    @pl.loop(0, NCHUNK)
    def _chunk(j):
        pltpu.sync_copy(src_hbm.at[wid, j], si)
        pltpu.sync_copy(dst_hbm.at[wid, j], di)
        pltpu.sync_copy(q_hbm.at[di], qr)
        pltpu.sync_copy(k_hbm.at[si], kr)

        @plsc.parallel_loop(0, C, step=L, unroll=2)
        def _rowgrp(e0):
            z = jnp.zeros((L,), jnp.float32)
            for i in range(L):
                acc = jnp.zeros((L,), jnp.float32)
                for g in range(D // L):
                    acc = acc + (qr[e0 + i, pl.ds(g * L, L)]
                                 * kr[e0 + i, pl.ds(g * L, L)])
                z = jnp.where(lanes == i, jnp.sum(acc) * inv_sqrt_d, z)
            al[pl.ds(e0, L)] = z

        @pl.loop(0, C, step=L)
        def _mxupd(e0):
            mx[...] = jnp.maximum(mx[...], al[pl.ds(e0, L)])

        pltpu.sync_copy(al, alpha_hbm.at[wid, j])

    pltpu.sync_copy(mx, mx_hbm.at[wid])


@functools.partial(
    pl.kernel,
    out_type=jax.ShapeDtypeStruct((NC, NPAD, D), jnp.float32),
    mesh=_mesh,
    compiler_params=_sc_params,
    scratch_types=[
        pltpu.VMEM((C,), jnp.int32),
        pltpu.VMEM((C,), jnp.int32),
        pltpu.VMEM((C, D), jnp.float32),
        pltpu.VMEM((C, D), jnp.float32),
        pltpu.VMEM((C,), jnp.float32),
        pltpu.VMEM((NW, L), jnp.float32),
        pltpu.VMEM_SHARED((NPAD, D), jnp.float32),
    ],
)
def _sc_msg_v(v_hbm, src_hbm, dst_hbm, alpha_hbm, mx_hbm, accv_hbm,
              si, di, vr, sv, al, mxv, accv_s):
    cid = lax.axis_index("c")
    sid = lax.axis_index("s")
    wid = sid * NC + cid
    lanes = lax.iota(jnp.int32, L)
    rows = [lanes + g * L for g in range(C // L)]

    @pl.loop(0, C)
    def _zrow(r):
        for g in range(D // L):
            sv[r, pl.ds(g * L, L)] = jnp.zeros((L,), jnp.float32)

    base = sid * RPT
    for t in range(RPT // C):
        pltpu.sync_copy(sv, accv_s.at[pl.ds(base + t * C, C)])

    pltpu.sync_copy(mx_hbm, mxv)

    def mx_body(i, mv):
        return jnp.maximum(mv, mxv[i, :])

    m = jnp.max(lax.fori_loop(1, NW, mx_body, mxv[0, :]))

    plsc.subcore_barrier()

    @pl.loop(0, NCHUNK)
    def _chunk(j):
        pltpu.sync_copy(src_hbm.at[wid, j], si)
        pltpu.sync_copy(dst_hbm.at[wid, j], di)
        pltpu.sync_copy(alpha_hbm.at[wid, j], al)
        pltpu.sync_copy(v_hbm.at[si], vr)

        @plsc.parallel_loop(0, C, step=L, unroll=2)
        def _rowgrp(e0):
            exv = jnp.exp(al[pl.ds(e0, L)] - m)
            for i in range(L):
                sc = exv[i]
                for g in range(D // L):
                    sv[e0 + i, pl.ds(g * L, L)] = vr[e0 + i, pl.ds(g * L, L)] * sc

        pltpu.sync_copy(sv, accv_s.at[di], add=True)

    plsc.subcore_barrier()
    pltpu.sync_copy(accv_s.at[pl.ds(base, RPT)], accv_hbm.at[cid, pl.ds(base, RPT)])


@functools.partial(
    pl.kernel,
    out_type=jax.ShapeDtypeStruct((NC, NPAD, D), jnp.float32),
    mesh=_mesh,
    compiler_params=_sc_params,
    scratch_types=[
        pltpu.VMEM((C,), jnp.int32),
        pltpu.VMEM((C,), jnp.float32),
        pltpu.VMEM((C, D), jnp.float32),
        pltpu.VMEM((NW, L), jnp.float32),
        pltpu.VMEM_SHARED((NPAD, D), jnp.float32),
    ],
)
def _sc_aux(dst_hbm, alpha_hbm, mx_hbm, aux_hbm, di, al, auxb, mxv, aux_s):
    cid = lax.axis_index("c")
    sid = lax.axis_index("s")
    wid = sid * NC + cid
    lanes = lax.iota(jnp.int32, L)
    rows = [lanes + g * L for g in range(C // L)]

    @pl.loop(0, C)
    def _zrow(r):
        for g in range(D // L):
            auxb[r, pl.ds(g * L, L)] = jnp.zeros((L,), jnp.float32)

    base = sid * RPT
    for t in range(RPT // C):
        pltpu.sync_copy(auxb, aux_s.at[pl.ds(base + t * C, C)])
    ones = jnp.ones((L,), jnp.float32)
    col1 = jnp.full((L,), 1, jnp.int32)
    col0 = jnp.full((L,), 0, jnp.int32)
    for g in range(C // L):
        plsc.store_scatter(auxb, [rows[g], col1], ones)

    pltpu.sync_copy(mx_hbm, mxv)

    def mx_body(i, mv):
        return jnp.maximum(mv, mxv[i, :])

    m = jnp.max(lax.fori_loop(1, NW, mx_body, mxv[0, :]))

    plsc.subcore_barrier()

    @pl.loop(0, NCHUNK)
    def _chunk(j):
        pltpu.sync_copy(dst_hbm.at[wid, j], di)
        pltpu.sync_copy(alpha_hbm.at[wid, j], al)
        for g in range(C // L):
            ex_g = jnp.exp(al[pl.ds(g * L, L)] - m)
            plsc.store_scatter(auxb, [rows[g], col0], ex_g)
        pltpu.sync_copy(auxb, aux_s.at[di], add=True)

    plsc.subcore_barrier()
    pltpu.sync_copy(aux_s.at[pl.ds(base, RPT)], aux_hbm.at[cid, pl.ds(base, RPT)])


@functools.partial(
    pl.kernel,
    out_type=jax.ShapeDtypeStruct((NC, NPAD, D), jnp.float32),
    mesh=_mesh,
    compiler_params=_sc_params,
    scratch_types=[
        pltpu.VMEM((C,), jnp.int32),
        pltpu.VMEM((C,), jnp.int32),
        pltpu.VMEM((C, D), jnp.float32),
        pltpu.VMEM_SHARED((NPAD, D), jnp.float32),
    ],
)
def _sc_msg_x1(x_hbm, src_hbm, dst_hbm, acc_hbm, si, di, xr, acc_s):
    cid = lax.axis_index("c")
    sid = lax.axis_index("s")
    wid = sid * NC + cid

    @pl.loop(0, C)
    def _zrow(r):
        for g in range(D // L):
            xr[r, pl.ds(g * L, L)] = jnp.zeros((L,), jnp.float32)

    base = sid * RPT
    for t in range(RPT // C):
        pltpu.sync_copy(xr, acc_s.at[pl.ds(base + t * C, C)])

    plsc.subcore_barrier()

    @pl.loop(0, NCHUNK)
    def _chunk(j):
        pltpu.sync_copy(src_hbm.at[wid, j], si)
        pltpu.sync_copy(dst_hbm.at[wid, j], di)
        pltpu.sync_copy(x_hbm.at[si], xr)
        pltpu.sync_copy(xr, acc_s.at[di], add=True)

    plsc.subcore_barrier()
    pltpu.sync_copy(acc_s.at[pl.ds(base, RPT)], acc_hbm.at[cid, pl.ds(base, RPT)])


@functools.partial(
    pl.kernel,
    out_type=jax.ShapeDtypeStruct((NC, NPAD, D), jnp.float32),
    mesh=_mesh,
    compiler_params=_sc_params,
    scratch_types=[
        pltpu.VMEM((C,), jnp.int32),
        pltpu.VMEM((C,), jnp.int32),
        pltpu.VMEM((C,), jnp.float32),
        pltpu.VMEM((C, D), jnp.float32),
        pltpu.VMEM((C, D), jnp.float32),
        pltpu.VMEM_SHARED((NPAD, D), jnp.float32),
    ],
)
def _sc_msg_x2(x_hbm, src_hbm, dst_hbm, ew_hbm, acc_hbm,
               si, di, ew, xr, sx, acc_s):
    cid = lax.axis_index("c")
    sid = lax.axis_index("s")
    wid = sid * NC + cid
    lanes = lax.iota(jnp.int32, L)
    rows = [lanes + g * L for g in range(C // L)]

    @pl.loop(0, C)
    def _zrow(r):
        for g in range(D // L):
            sx[r, pl.ds(g * L, L)] = jnp.zeros((L,), jnp.float32)

    base = sid * RPT
    for t in range(RPT // C):
        pltpu.sync_copy(sx, acc_s.at[pl.ds(base + t * C, C)])

    plsc.subcore_barrier()

    @pl.loop(0, NCHUNK)
    def _chunk(j):
        pltpu.sync_copy(src_hbm.at[wid, j], si)
        pltpu.sync_copy(dst_hbm.at[wid, j], di)
        pltpu.sync_copy(ew_hbm.at[wid, j], ew)
        pltpu.sync_copy(x_hbm.at[si], xr)

        @plsc.parallel_loop(0, C, step=L, unroll=2)
        def _rowgrp(e0):
            ewv = ew[pl.ds(e0, L)]
            for i in range(L):
                sc = ewv[i]
                for g in range(D // L):
                    sx[e0 + i, pl.ds(g * L, L)] = xr[e0 + i, pl.ds(g * L, L)] * sc

        pltpu.sync_copy(sx, acc_s.at[di], add=True)

    plsc.subcore_barrier()
    pltpu.sync_copy(acc_s.at[pl.ds(base, RPT)], acc_hbm.at[cid, pl.ds(base, RPT)])


def _tc_qkv_body(x_ref, w_ref, b_ref, q_ref, k_ref, v_ref):
    y = jnp.dot(x_ref[...], w_ref[...], preferred_element_type=jnp.float32)
    y = y + b_ref[...]
    q_ref[...] = y[:, :D]
    k_ref[...] = y[:, D:2 * D]
    v_ref[...] = y[:, 2 * D:]


def _tc_qkv(xp, W3, b3):
    blk = 1024
    grid = (NPAD // blk,)
    return pl.pallas_call(
        _tc_qkv_body,
        grid=grid,
        in_specs=[
            pl.BlockSpec((blk, D), lambda i: (i, 0)),
            pl.BlockSpec((D, 3 * D), lambda i: (0, 0)),
            pl.BlockSpec((1, 3 * D), lambda i: (0, 0)),
        ],
        out_specs=[
            pl.BlockSpec((blk, D), lambda i: (i, 0)),
            pl.BlockSpec((blk, D), lambda i: (i, 0)),
            pl.BlockSpec((blk, D), lambda i: (i, 0)),
        ],
        out_shape=[jax.ShapeDtypeStruct((NPAD, D), jnp.float32)] * 3,
    )(xp, W3, b3)


def _tc_final_body(x_ref, accv_ref, aux_ref, accx_ref, accxw_ref,
                   wskip_ref, bskip_ref, wsl_ref, bsl_ref, wsr_ref,
                   wrel_ref, brel_ref, wroot_ref,
                   wqkv_ref, bqkv_ref, wproj_ref, bproj_ref,
                   wfc_ref, bfc_ref, out_ref):
    x = x_ref[...]
    accv = accv_ref[0] + accv_ref[1]
    aux = aux_ref[0] + aux_ref[1]
    denom = aux[:, 0:1]
    deg_c = jnp.maximum(aux[:, 1:2], 1.0)
    accx = accx_ref[0] + accx_ref[1]
    accxw = accxw_ref[0] + accxw_ref[1]

    def mm(a, w):
        return jnp.dot(a, w, preferred_element_type=jnp.float32)

    x_gc1 = accv / (denom + 1e-16) + mm(x, wskip_ref[...]) + bskip_ref[...]
    x_gc2 = mm(accx / deg_c, wsl_ref[...]) + bsl_ref[...] + mm(x, wsr_ref[...])
    x_gc3 = mm(accxw / deg_c, wrel_ref[...]) + brel_ref[...] + mm(x, wroot_ref[...])

    toks = [x_gc1, x_gc2, x_gc3]
    wqkv = wqkv_ref[...]
    bqkv = bqkv_ref[...]
    qs, ks, vs = [], [], []
    for t in toks:
        qkv = mm(t, wqkv) + bqkv
        qs.append(qkv[:, :D])
        ks.append(qkv[:, D:2 * D])
        vs.append(qkv[:, 2 * D:])

    scale = HD ** -0.5
    outs = []
    for i in range(3):
        halves = []
        for h in range(H):
            sl = slice(h * HD, (h + 1) * HD)
            s = [jnp.sum(qs[i][:, sl] * ks[j][:, sl], axis=1, keepdims=True)
                 * scale for j in range(3)]
            m = jnp.maximum(jnp.maximum(s[0], s[1]), s[2])
            e = [jnp.exp(sj - m) for sj in s]
            den = e[0] + e[1] + e[2]
            halves.append(sum(e[j] / den * vs[j][:, sl] for j in range(3)))
        out_i = jnp.concatenate(halves, axis=1)
        outs.append(mm(out_i, wproj_ref[...]) + bproj_ref[...])

    wfc = wfc_ref[...]
    final = mm(outs[0], wfc[:D, :]) + mm(outs[1], wfc[D:2 * D, :]) \
        + mm(outs[2], wfc[2 * D:, :]) + bfc_ref[...]
    out_ref[...] = final


def _tc_final(xp, accv, aux, accx, accxw, Wskip, bskip, Wsl, bsl, Wsr,
              Wrel, brel, Wroot, Wqkv, bqkv, Wproj, bproj, Wfc, bfc):
    bskip, bsl, brel, bproj, bfc = (
        b.reshape(1, D) for b in (bskip, bsl, brel, bproj, bfc))
    bqkv = bqkv.reshape(1, 3 * D)
    blk = 1024
    grid = (NPAD // blk,)

    def row_spec(minor):
        return pl.BlockSpec((NC, blk, minor), lambda i: (0, i, 0))

    def w_spec(r, c):
        return pl.BlockSpec((r, c), lambda i: (0, 0))

    return pl.pallas_call(
        _tc_final_body,
        grid=grid,
        in_specs=[
            pl.BlockSpec((blk, D), lambda i: (i, 0)),
            row_spec(D), row_spec(D), row_spec(D), row_spec(D),
            w_spec(D, D), w_spec(1, D),
            w_spec(D, D), w_spec(1, D), w_spec(D, D),
            w_spec(D, D), w_spec(1, D), w_spec(D, D),
            w_spec(D, 3 * D), w_spec(1, 3 * D),
            w_spec(D, D), w_spec(1, D),
            w_spec(3 * D, D), w_spec(1, D),
        ],
        out_specs=pl.BlockSpec((blk, D), lambda i: (i, 0)),
        out_shape=jax.ShapeDtypeStruct((NPAD, D), jnp.float32),
    )(xp, accv, aux, accx, accxw, Wskip, bskip, Wsl, bsl, Wsr,
      Wrel, brel, Wroot, Wqkv, bqkv, Wproj, bproj, Wfc, bfc)


def kernel(x, edge_index, edge_weight, Wq, bq, Wk, bk, Wv, bv, Wskip, bskip,
           Wsl, bsl, Wsr, Wrel, brel, Wroot, Wqkv, bqkv, Wproj, bproj,
           Wfc, bfc):
    xp = jnp.pad(x, ((0, NPAD - N), (0, 0)))
    pad_e = EPAD - E
    src = jnp.concatenate([edge_index[0], jnp.full((pad_e,), N, jnp.int32)])
    dst = jnp.concatenate([edge_index[1], jnp.full((pad_e,), N, jnp.int32)])
    ew = jnp.concatenate([edge_weight, jnp.zeros((pad_e,), jnp.float32)])
    srcr = src.reshape(NW, NCHUNK, C)
    dstr = dst.reshape(NW, NCHUNK, C)
    ewr = ew.reshape(NW, NCHUNK, C)

    W3 = jnp.concatenate([Wq, Wk, Wv], axis=1)
    b3 = jnp.concatenate([bq, bk, bv]).reshape(1, 3 * D)

    q, k, v = _tc_qkv(xp, W3, b3)
    alpha, mx = _sc_alpha(q, k, srcr, dstr)
    accv = _sc_msg_v(v, srcr, dstr, alpha, mx)
    aux = _sc_aux(dstr, alpha, mx)
    accx = _sc_msg_x1(xp, srcr, dstr)
    accxw = _sc_msg_x2(xp, srcr, dstr, ewr)

    out = _tc_final(xp, accv, aux, accx, accxw, Wskip, bskip, Wsl, bsl, Wsr,
                    Wrel, brel, Wroot, Wqkv, bqkv, Wproj, bproj, Wfc, bfc)
    return out[:N]


# speedup vs baseline: 4.2357x; 1.1331x over previous
import dataclasses
import functools

import jax
import jax.numpy as jnp
from jax import lax
from jax.experimental import pallas as pl
from jax.experimental.pallas import tpu as pltpu
from jax.experimental.pallas import tpu_sc as plsc

N = 10000
E = 320000
D = 128
H = 2
HD = D // H

NC = 2
NS = 16
L = 16
NW = NC * NS

C = 128
NCHUNK = 80
EPW = C * NCHUNK
EPAD = EPW * NW
NPAD = 10240
RPT = NPAD // NS

_mesh = plsc.VectorSubcoreMesh(core_axis_name="c", subcore_axis_name="s")

_sc_params = pltpu.CompilerParams()
if "needs_layout_passes" in pltpu.CompilerParams.__dataclass_fields__:
    _sc_params = dataclasses.replace(_sc_params, needs_layout_passes=False)


def _wid():
    return lax.axis_index("s") * NC + lax.axis_index("c")


@functools.partial(
    pl.kernel,
    out_type=(
        jax.ShapeDtypeStruct((NW, NCHUNK, C), jnp.float32),
        jax.ShapeDtypeStruct((NW, L), jnp.float32),
    ),
    mesh=_mesh,
    compiler_params=_sc_params,
    scratch_types=[
        pltpu.VMEM((C,), jnp.int32),
        pltpu.VMEM((C,), jnp.int32),
        pltpu.VMEM((C,), jnp.int32),
        pltpu.VMEM((C,), jnp.int32),
        pltpu.VMEM((C, D), jnp.float32),
        pltpu.VMEM((C, D), jnp.float32),
        pltpu.VMEM((C, D), jnp.float32),
        pltpu.VMEM((C, D), jnp.float32),
        pltpu.VMEM((C,), jnp.float32),
        pltpu.VMEM((L,), jnp.float32),
        pltpu.SemaphoreType.DMA,
        pltpu.SemaphoreType.DMA,
    ],
)
def _sc_alpha(q_hbm, k_hbm, src_hbm, dst_hbm, alpha_hbm, mx_hbm,
              si0, di0, si1, di1, qr0, kr0, qr1, kr1, al, mx, sem0, sem1):
    wid = _wid()
    inv_sqrt_d = 1.0 / (D ** 0.5)
    lanes = lax.iota(jnp.int32, L)
    mx[...] = jnp.full((L,), -1e30, jnp.float32)

    pltpu.sync_copy(src_hbm.at[wid, 0], si0)
    pltpu.sync_copy(dst_hbm.at[wid, 0], di0)
    pltpu.async_copy(q_hbm.at[di0], qr0, sem0)
    pltpu.async_copy(k_hbm.at[si0], kr0, sem0)

    def compute(qr, kr):
        @plsc.parallel_loop(0, C, step=L, unroll=2)
        def _rowgrp(e0):
            z = jnp.zeros((L,), jnp.float32)
            for i in range(L):
                acc = jnp.zeros((L,), jnp.float32)
                for g in range(D // L):
                    acc = acc + (qr[e0 + i, pl.ds(g * L, L)]
                                 * kr[e0 + i, pl.ds(g * L, L)])
                z = jnp.where(lanes == i, jnp.sum(acc) * inv_sqrt_d, z)
            al[pl.ds(e0, L)] = z

        @pl.loop(0, C, step=L)
        def _mxupd(e0):
            mx[...] = jnp.maximum(mx[...], al[pl.ds(e0, L)])

    @pl.loop(0, NCHUNK, step=2)
    def _chunk(j):
        pltpu.sync_copy(src_hbm.at[wid, j + 1], si1)
        pltpu.sync_copy(dst_hbm.at[wid, j + 1], di1)
        pltpu.async_copy(q_hbm.at[di1], qr1, sem1)
        pltpu.async_copy(k_hbm.at[si1], kr1, sem1)
        pltpu.make_async_copy(q_hbm.at[di0], qr0, sem0).wait()
        pltpu.make_async_copy(k_hbm.at[si0], kr0, sem0).wait()
        compute(qr0, kr0)
        pltpu.sync_copy(al, alpha_hbm.at[wid, j])

        @pl.when(j + 2 < NCHUNK)
        def _pf():
            pltpu.sync_copy(src_hbm.at[wid, j + 2], si0)
            pltpu.sync_copy(dst_hbm.at[wid, j + 2], di0)
            pltpu.async_copy(q_hbm.at[di0], qr0, sem0)
            pltpu.async_copy(k_hbm.at[si0], kr0, sem0)

        pltpu.make_async_copy(q_hbm.at[di1], qr1, sem1).wait()
        pltpu.make_async_copy(k_hbm.at[si1], kr1, sem1).wait()
        compute(qr1, kr1)
        pltpu.sync_copy(al, alpha_hbm.at[wid, j + 1])

    pltpu.sync_copy(mx, mx_hbm.at[wid])


@functools.partial(
    pl.kernel,
    out_type=jax.ShapeDtypeStruct((NC, NPAD, D), jnp.float32),
    mesh=_mesh,
    compiler_params=_sc_params,
    scratch_types=[
        pltpu.VMEM((C,), jnp.int32),
        pltpu.VMEM((C,), jnp.int32),
        pltpu.VMEM((C,), jnp.float32),
        pltpu.VMEM((C,), jnp.int32),
        pltpu.VMEM((C,), jnp.int32),
        pltpu.VMEM((C,), jnp.float32),
        pltpu.VMEM((C, D), jnp.float32),
        pltpu.VMEM((C, D), jnp.float32),
        pltpu.VMEM((NW, L), jnp.float32),
        pltpu.SemaphoreType.DMA,
        pltpu.SemaphoreType.DMA,
        pltpu.VMEM_SHARED((NPAD, D), jnp.float32),
    ],
)
def _sc_msg_v(v_hbm, src_hbm, dst_hbm, alpha_hbm, mx_hbm, accv_hbm,
              si0, di0, al0, si1, di1, al1, vr0, vr1, mxv, sem0, sem1,
              accv_s):
    cid = lax.axis_index("c")
    sid = lax.axis_index("s")
    wid = sid * NC + cid

    @pl.loop(0, C)
    def _zrow(r):
        for g in range(D // L):
            vr0[r, pl.ds(g * L, L)] = jnp.zeros((L,), jnp.float32)

    base = sid * RPT
    for t in range(RPT // C):
        pltpu.sync_copy(vr0, accv_s.at[pl.ds(base + t * C, C)])

    pltpu.sync_copy(mx_hbm, mxv)

    def mx_body(i, mv):
        return jnp.maximum(mv, mxv[i, :])

    m = jnp.max(lax.fori_loop(1, NW, mx_body, mxv[0, :]))

    plsc.subcore_barrier()

    pltpu.sync_copy(src_hbm.at[wid, 0], si0)
    pltpu.sync_copy(dst_hbm.at[wid, 0], di0)
    pltpu.sync_copy(alpha_hbm.at[wid, 0], al0)
    pltpu.async_copy(v_hbm.at[si0], vr0, sem0)

    def scale(vr, al):
        @plsc.parallel_loop(0, C, step=L, unroll=2)
        def _rowgrp(e0):
            exv = jnp.exp(al[pl.ds(e0, L)] - m)
            for i in range(L):
                sc = exv[i]
                for g in range(D // L):
                    vr[e0 + i, pl.ds(g * L, L)] = vr[e0 + i, pl.ds(g * L, L)] * sc

    @pl.loop(0, NCHUNK, step=2)
    def _chunk(j):
        pltpu.sync_copy(src_hbm.at[wid, j + 1], si1)
        pltpu.sync_copy(dst_hbm.at[wid, j + 1], di1)
        pltpu.sync_copy(alpha_hbm.at[wid, j + 1], al1)
        pltpu.async_copy(v_hbm.at[si1], vr1, sem1)
        pltpu.make_async_copy(v_hbm.at[si0], vr0, sem0).wait()
        scale(vr0, al0)
        pltpu.sync_copy(vr0, accv_s.at[di0], add=True)

        @pl.when(j + 2 < NCHUNK)
        def _pf():
            pltpu.sync_copy(src_hbm.at[wid, j + 2], si0)
            pltpu.sync_copy(dst_hbm.at[wid, j + 2], di0)
            pltpu.sync_copy(alpha_hbm.at[wid, j + 2], al0)
            pltpu.async_copy(v_hbm.at[si0], vr0, sem0)

        pltpu.make_async_copy(v_hbm.at[si1], vr1, sem1).wait()
        scale(vr1, al1)
        pltpu.sync_copy(vr1, accv_s.at[di1], add=True)

    plsc.subcore_barrier()
    pltpu.sync_copy(accv_s.at[pl.ds(base, RPT)], accv_hbm.at[cid, pl.ds(base, RPT)])


@functools.partial(
    pl.kernel,
    out_type=jax.ShapeDtypeStruct((NC, NPAD, D), jnp.float32),
    mesh=_mesh,
    compiler_params=_sc_params,
    scratch_types=[
        pltpu.VMEM((C,), jnp.int32),
        pltpu.VMEM((C,), jnp.float32),
        pltpu.VMEM((C, D), jnp.float32),
        pltpu.VMEM((NW, L), jnp.float32),
        pltpu.VMEM_SHARED((NPAD, D), jnp.float32),
    ],
)
def _sc_aux(dst_hbm, alpha_hbm, mx_hbm, aux_hbm, di, al, auxb, mxv, aux_s):
    cid = lax.axis_index("c")
    sid = lax.axis_index("s")
    wid = sid * NC + cid
    lanes = lax.iota(jnp.int32, L)
    rows = [lanes + g * L for g in range(C // L)]

    @pl.loop(0, C)
    def _zrow(r):
        for g in range(D // L):
            auxb[r, pl.ds(g * L, L)] = jnp.zeros((L,), jnp.float32)

    base = sid * RPT
    for t in range(RPT // C):
        pltpu.sync_copy(auxb, aux_s.at[pl.ds(base + t * C, C)])
    ones = jnp.ones((L,), jnp.float32)
    col1 = jnp.full((L,), 1, jnp.int32)
    col0 = jnp.full((L,), 0, jnp.int32)
    for g in range(C // L):
        plsc.store_scatter(auxb, [rows[g], col1], ones)

    pltpu.sync_copy(mx_hbm, mxv)

    def mx_body(i, mv):
        return jnp.maximum(mv, mxv[i, :])

    m = jnp.max(lax.fori_loop(1, NW, mx_body, mxv[0, :]))

    plsc.subcore_barrier()

    @pl.loop(0, NCHUNK)
    def _chunk(j):
        pltpu.sync_copy(dst_hbm.at[wid, j], di)
        pltpu.sync_copy(alpha_hbm.at[wid, j], al)
        for g in range(C // L):
            ex_g = jnp.exp(al[pl.ds(g * L, L)] - m)
            plsc.store_scatter(auxb, [rows[g], col0], ex_g)
        pltpu.sync_copy(auxb, aux_s.at[di], add=True)

    plsc.subcore_barrier()
    pltpu.sync_copy(aux_s.at[pl.ds(base, RPT)], aux_hbm.at[cid, pl.ds(base, RPT)])


@functools.partial(
    pl.kernel,
    out_type=jax.ShapeDtypeStruct((NC, NPAD, D), jnp.float32),
    mesh=_mesh,
    compiler_params=_sc_params,
    scratch_types=[
        pltpu.VMEM((C,), jnp.int32),
        pltpu.VMEM((C,), jnp.int32),
        pltpu.VMEM((C,), jnp.int32),
        pltpu.VMEM((C,), jnp.int32),
        pltpu.VMEM((C, D), jnp.float32),
        pltpu.VMEM((C, D), jnp.float32),
        pltpu.SemaphoreType.DMA,
        pltpu.SemaphoreType.DMA,
        pltpu.VMEM_SHARED((NPAD, D), jnp.float32),
    ],
)
def _sc_msg_x1(x_hbm, src_hbm, dst_hbm, acc_hbm,
               si0, di0, si1, di1, xr0, xr1, sem0, sem1, acc_s):
    cid = lax.axis_index("c")
    sid = lax.axis_index("s")
    wid = sid * NC + cid

    @pl.loop(0, C)
    def _zrow(r):
        for g in range(D // L):
            xr0[r, pl.ds(g * L, L)] = jnp.zeros((L,), jnp.float32)

    base = sid * RPT
    for t in range(RPT // C):
        pltpu.sync_copy(xr0, acc_s.at[pl.ds(base + t * C, C)])

    plsc.subcore_barrier()

    pltpu.sync_copy(src_hbm.at[wid, 0], si0)
    pltpu.sync_copy(dst_hbm.at[wid, 0], di0)
    pltpu.async_copy(x_hbm.at[si0], xr0, sem0)

    @pl.loop(0, NCHUNK, step=2)
    def _chunk(j):
        pltpu.sync_copy(src_hbm.at[wid, j + 1], si1)
        pltpu.sync_copy(dst_hbm.at[wid, j + 1], di1)
        pltpu.async_copy(x_hbm.at[si1], xr1, sem1)
        pltpu.make_async_copy(x_hbm.at[si0], xr0, sem0).wait()
        pltpu.sync_copy(xr0, acc_s.at[di0], add=True)

        @pl.when(j + 2 < NCHUNK)
        def _pf():
            pltpu.sync_copy(src_hbm.at[wid, j + 2], si0)
            pltpu.sync_copy(dst_hbm.at[wid, j + 2], di0)
            pltpu.async_copy(x_hbm.at[si0], xr0, sem0)

        pltpu.make_async_copy(x_hbm.at[si1], xr1, sem1).wait()
        pltpu.sync_copy(xr1, acc_s.at[di1], add=True)

    plsc.subcore_barrier()
    pltpu.sync_copy(acc_s.at[pl.ds(base, RPT)], acc_hbm.at[cid, pl.ds(base, RPT)])


@functools.partial(
    pl.kernel,
    out_type=jax.ShapeDtypeStruct((NC, NPAD, D), jnp.float32),
    mesh=_mesh,
    compiler_params=_sc_params,
    scratch_types=[
        pltpu.VMEM((C,), jnp.int32),
        pltpu.VMEM((C,), jnp.int32),
        pltpu.VMEM((C,), jnp.float32),
        pltpu.VMEM((C,), jnp.int32),
        pltpu.VMEM((C,), jnp.int32),
        pltpu.VMEM((C,), jnp.float32),
        pltpu.VMEM((C, D), jnp.float32),
        pltpu.VMEM((C, D), jnp.float32),
        pltpu.SemaphoreType.DMA,
        pltpu.SemaphoreType.DMA,
        pltpu.VMEM_SHARED((NPAD, D), jnp.float32),
    ],
)
def _sc_msg_x2(x_hbm, src_hbm, dst_hbm, ew_hbm, acc_hbm,
               si0, di0, ew0, si1, di1, ew1, xr0, xr1, sem0, sem1, acc_s):
    cid = lax.axis_index("c")
    sid = lax.axis_index("s")
    wid = sid * NC + cid

    @pl.loop(0, C)
    def _zrow(r):
        for g in range(D // L):
            xr0[r, pl.ds(g * L, L)] = jnp.zeros((L,), jnp.float32)

    base = sid * RPT
    for t in range(RPT // C):
        pltpu.sync_copy(xr0, acc_s.at[pl.ds(base + t * C, C)])

    plsc.subcore_barrier()

    pltpu.sync_copy(src_hbm.at[wid, 0], si0)
    pltpu.sync_copy(dst_hbm.at[wid, 0], di0)
    pltpu.sync_copy(ew_hbm.at[wid, 0], ew0)
    pltpu.async_copy(x_hbm.at[si0], xr0, sem0)

    def scale(xr, ew):
        @plsc.parallel_loop(0, C, step=L, unroll=2)
        def _rowgrp(e0):
            ewv = ew[pl.ds(e0, L)]
            for i in range(L):
                sc = ewv[i]
                for g in range(D // L):
                    xr[e0 + i, pl.ds(g * L, L)] = xr[e0 + i, pl.ds(g * L, L)] * sc

    @pl.loop(0, NCHUNK, step=2)
    def _chunk(j):
        pltpu.sync_copy(src_hbm.at[wid, j + 1], si1)
        pltpu.sync_copy(dst_hbm.at[wid, j + 1], di1)
        pltpu.sync_copy(ew_hbm.at[wid, j + 1], ew1)
        pltpu.async_copy(x_hbm.at[si1], xr1, sem1)
        pltpu.make_async_copy(x_hbm.at[si0], xr0, sem0).wait()
        scale(xr0, ew0)
        pltpu.sync_copy(xr0, acc_s.at[di0], add=True)

        @pl.when(j + 2 < NCHUNK)
        def _pf():
            pltpu.sync_copy(src_hbm.at[wid, j + 2], si0)
            pltpu.sync_copy(dst_hbm.at[wid, j + 2], di0)
            pltpu.sync_copy(ew_hbm.at[wid, j + 2], ew0)
            pltpu.async_copy(x_hbm.at[si0], xr0, sem0)

        pltpu.make_async_copy(x_hbm.at[si1], xr1, sem1).wait()
        scale(xr1, ew1)
        pltpu.sync_copy(xr1, acc_s.at[di1], add=True)

    plsc.subcore_barrier()
    pltpu.sync_copy(acc_s.at[pl.ds(base, RPT)], acc_hbm.at[cid, pl.ds(base, RPT)])


def _tc_qkv_body(x_ref, w_ref, b_ref, q_ref, k_ref, v_ref):
    y = jnp.dot(x_ref[...], w_ref[...], preferred_element_type=jnp.float32)
    y = y + b_ref[...]
    q_ref[...] = y[:, :D]
    k_ref[...] = y[:, D:2 * D]
    v_ref[...] = y[:, 2 * D:]


def _tc_qkv(xp, W3, b3):
    blk = 1024
    grid = (NPAD // blk,)
    return pl.pallas_call(
        _tc_qkv_body,
        grid=grid,
        in_specs=[
            pl.BlockSpec((blk, D), lambda i: (i, 0)),
            pl.BlockSpec((D, 3 * D), lambda i: (0, 0)),
            pl.BlockSpec((1, 3 * D), lambda i: (0, 0)),
        ],
        out_specs=[
            pl.BlockSpec((blk, D), lambda i: (i, 0)),
            pl.BlockSpec((blk, D), lambda i: (i, 0)),
            pl.BlockSpec((blk, D), lambda i: (i, 0)),
        ],
        out_shape=[jax.ShapeDtypeStruct((NPAD, D), jnp.float32)] * 3,
    )(xp, W3, b3)


def _tc_final_body(x_ref, accv_ref, aux_ref, accx_ref, accxw_ref,
                   wskip_ref, bskip_ref, wsl_ref, bsl_ref, wsr_ref,
                   wrel_ref, brel_ref, wroot_ref,
                   wqkv_ref, bqkv_ref, wproj_ref, bproj_ref,
                   wfc_ref, bfc_ref, out_ref):
    x = x_ref[...]
    accv = accv_ref[0] + accv_ref[1]
    aux = aux_ref[0] + aux_ref[1]
    denom = aux[:, 0:1]
    deg_c = jnp.maximum(aux[:, 1:2], 1.0)
    accx = accx_ref[0] + accx_ref[1]
    accxw = accxw_ref[0] + accxw_ref[1]

    def mm(a, w):
        return jnp.dot(a, w, preferred_element_type=jnp.float32)

    x_gc1 = accv / (denom + 1e-16) + mm(x, wskip_ref[...]) + bskip_ref[...]
    x_gc2 = mm(accx / deg_c, wsl_ref[...]) + bsl_ref[...] + mm(x, wsr_ref[...])
    x_gc3 = mm(accxw / deg_c, wrel_ref[...]) + brel_ref[...] + mm(x, wroot_ref[...])

    toks = [x_gc1, x_gc2, x_gc3]
    wqkv = wqkv_ref[...]
    bqkv = bqkv_ref[...]
    qs, ks, vs = [], [], []
    for t in toks:
        qkv = mm(t, wqkv) + bqkv
        qs.append(qkv[:, :D])
        ks.append(qkv[:, D:2 * D])
        vs.append(qkv[:, 2 * D:])

    scale = HD ** -0.5
    outs = []
    for i in range(3):
        halves = []
        for h in range(H):
            sl = slice(h * HD, (h + 1) * HD)
            s = [jnp.sum(qs[i][:, sl] * ks[j][:, sl], axis=1, keepdims=True)
                 * scale for j in range(3)]
            m = jnp.maximum(jnp.maximum(s[0], s[1]), s[2])
            e = [jnp.exp(sj - m) for sj in s]
            den = e[0] + e[1] + e[2]
            halves.append(sum(e[j] / den * vs[j][:, sl] for j in range(3)))
        out_i = jnp.concatenate(halves, axis=1)
        outs.append(mm(out_i, wproj_ref[...]) + bproj_ref[...])

    wfc = wfc_ref[...]
    final = mm(outs[0], wfc[:D, :]) + mm(outs[1], wfc[D:2 * D, :]) \
        + mm(outs[2], wfc[2 * D:, :]) + bfc_ref[...]
    out_ref[...] = final


def _tc_final(xp, accv, aux, accx, accxw, Wskip, bskip, Wsl, bsl, Wsr,
              Wrel, brel, Wroot, Wqkv, bqkv, Wproj, bproj, Wfc, bfc):
    bskip, bsl, brel, bproj, bfc = (
        b.reshape(1, D) for b in (bskip, bsl, brel, bproj, bfc))
    bqkv = bqkv.reshape(1, 3 * D)
    blk = 1024
    grid = (NPAD // blk,)

    def row_spec(minor):
        return pl.BlockSpec((NC, blk, minor), lambda i: (0, i, 0))

    def w_spec(r, c):
        return pl.BlockSpec((r, c), lambda i: (0, 0))

    return pl.pallas_call(
        _tc_final_body,
        grid=grid,
        in_specs=[
            pl.BlockSpec((blk, D), lambda i: (i, 0)),
            row_spec(D), row_spec(D), row_spec(D), row_spec(D),
            w_spec(D, D), w_spec(1, D),
            w_spec(D, D), w_spec(1, D), w_spec(D, D),
            w_spec(D, D), w_spec(1, D), w_spec(D, D),
            w_spec(D, 3 * D), w_spec(1, 3 * D),
            w_spec(D, D), w_spec(1, D),
            w_spec(3 * D, D), w_spec(1, D),
        ],
        out_specs=pl.BlockSpec((blk, D), lambda i: (i, 0)),
        out_shape=jax.ShapeDtypeStruct((NPAD, D), jnp.float32),
    )(xp, accv, aux, accx, accxw, Wskip, bskip, Wsl, bsl, Wsr,
      Wrel, brel, Wroot, Wqkv, bqkv, Wproj, bproj, Wfc, bfc)


def kernel(x, edge_index, edge_weight, Wq, bq, Wk, bk, Wv, bv, Wskip, bskip,
           Wsl, bsl, Wsr, Wrel, brel, Wroot, Wqkv, bqkv, Wproj, bproj,
           Wfc, bfc):
    xp = jnp.pad(x, ((0, NPAD - N), (0, 0)))
    pad_e = EPAD - E
    src = jnp.concatenate([edge_index[0], jnp.full((pad_e,), N, jnp.int32)])
    dst = jnp.concatenate([edge_index[1], jnp.full((pad_e,), N, jnp.int32)])
    ew = jnp.concatenate([edge_weight, jnp.zeros((pad_e,), jnp.float32)])
    srcr = src.reshape(NW, NCHUNK, C)
    dstr = dst.reshape(NW, NCHUNK, C)
    ewr = ew.reshape(NW, NCHUNK, C)

    W3 = jnp.concatenate([Wq, Wk, Wv], axis=1)
    b3 = jnp.concatenate([bq, bk, bv]).reshape(1, 3 * D)

    q, k, v = _tc_qkv(xp, W3, b3)
    alpha, mx = _sc_alpha(q, k, srcr, dstr)
    accv = _sc_msg_v(v, srcr, dstr, alpha, mx)
    aux = _sc_aux(dstr, alpha, mx)
    accx = _sc_msg_x1(xp, srcr, dstr)
    accxw = _sc_msg_x2(xp, srcr, dstr, ewr)

    out = _tc_final(xp, accv, aux, accx, accxw, Wskip, bskip, Wsl, bsl, Wsr,
                    Wrel, brel, Wroot, Wqkv, bqkv, Wproj, bproj, Wfc, bfc)
    return out[:N]
